# baseline (speedup 1.0000x reference)
import functools

import jax
import jax.numpy as jnp
from jax import lax
from jax.experimental import pallas as pl
from jax.experimental.pallas import tpu as pltpu
from jax.experimental.pallas import tpu_sc as plsc

B, N, F, K = 2, 10000, 128, 16
KP1 = K + 1
NODES = B * N

_info = plsc.get_sparse_core_info()
NC, NS = _info.num_cores, _info.num_subcores
NW = NC * NS

NPS = 8
IPS = NPS * K
RPS = NPS * KP1
STEPS = NODES // NPS
STEPS_LO = STEPS // NW
EXTRA = STEPS - STEPS_LO * NW
MAX_STEPS_W = STEPS_LO + 1
NBUF = 4
IBUF = 160

NBLK = (STEPS_LO - 2 - NBUF) // NBUF

_mesh = plsc.VectorSubcoreMesh(core_axis_name="c", subcore_axis_name="s")


@functools.partial(
    pl.kernel,
    mesh=_mesh,
    out_type=jax.ShapeDtypeStruct((NODES * KP1, F), jnp.float32),
    scratch_types=[
        pltpu.VMEM((MAX_STEPS_W * IPS,), jnp.int32),
        pltpu.VMEM((NBUF * IBUF,), jnp.int32),
        pltpu.VMEM((NBUF, RPS, F), jnp.float32),
        pltpu.SemaphoreType.DMA,
        pltpu.SemaphoreType.DMA,
        pltpu.SemaphoreType.DMA,
        pltpu.SemaphoreType.DMA,
        pltpu.SemaphoreType.DMA,
        pltpu.SemaphoreType.DMA,
        pltpu.SemaphoreType.DMA,
        pltpu.SemaphoreType.DMA,
    ],
)
def _gather_kernel(inp_hbm, inds_hbm, out_hbm, inds_v, idx_v, rows_v,
                   sg0, sg1, sg2, sg3, sw0, sw1, sw2, sw3):
    sem_g = (sg0, sg1, sg2, sg3)
    sem_w = (sw0, sw1, sw2, sw3)

    wid = lax.axis_index("s") * NC + lax.axis_index("c")
    nsteps = STEPS_LO + jnp.where(wid < EXTRA, 1, 0)
    s0 = STEPS_LO * wid + jnp.minimum(wid, EXTRA)

    stage_s0 = jnp.minimum(s0, STEPS - MAX_STEPS_W)
    pltpu.sync_copy(
        inds_hbm.at[pl.ds(stage_s0 * IPS, MAX_STEPS_W * IPS)], inds_v)
    ioff0 = (s0 - stage_s0) * IPS

    zeros = jnp.zeros((16,), jnp.int32)

    def build_fire(i, b):
        s = s0 + i
        node0 = s * NPS
        base = jnp.where(node0 >= N, N, 0).astype(jnp.int32)
        ioff = ioff0 + i * IPS
        boff = b * IBUF
        for g in range(NPS):
            idx_v[pl.ds(boff + g * KP1, K)] = (
                inds_v[pl.ds(ioff + g * K, K)] + base)
            idx_v[pl.ds(boff + g * KP1 + K, 16)] = zeros + (node0 + g)
        pltpu.async_copy(
            inp_hbm.at[idx_v.at[pl.ds(boff, 64)]],
            rows_v.at[b].at[pl.ds(0, 64)], sem_g[b])
        pltpu.async_copy(
            inp_hbm.at[idx_v.at[pl.ds(boff + 64, RPS - 64)]],
            rows_v.at[b].at[pl.ds(64, RPS - 64)], sem_g[b])

    def wait_gather(b):
        boff = b * IBUF
        pltpu.make_async_copy(
            inp_hbm.at[0].at[idx_v.at[pl.ds(boff, 64)]],
            rows_v.at[b].at[pl.ds(0, 64)], sem_g[b]).wait()
        pltpu.make_async_copy(
            inp_hbm.at[0].at[idx_v.at[pl.ds(boff + 64, RPS - 64)]],
            rows_v.at[b].at[pl.ds(64, RPS - 64)], sem_g[b]).wait()

    def fire_write(i, b):
        row0 = (s0 + i) * RPS
        pltpu.async_copy(rows_v.at[b], out_hbm.at[pl.ds(row0, RPS)],
                         sem_w[b])

    def wait_write(b):
        pltpu.make_async_copy(rows_v.at[b], out_hbm.at[pl.ds(0, RPS)],
                              sem_w[b]).wait()

    build_fire(0, 0)
    for i in range(1, NBUF):
        build_fire(i, i)
        wait_gather(i - 1)
        fire_write(i - 1, i - 1)

    def block(p, carry):
        i0 = NBUF + p * NBUF
        for j in range(NBUF):
            i = i0 + j
            bp = (j + NBUF - 1) % NBUF
            wait_write(j)
            build_fire(i, j)
            wait_gather(bp)
            fire_write(i - 1, bp)
        return carry

    lax.fori_loop(0, NBLK, block, 0)

    i_a, i_b = NBUF + 4 * NBLK, NBUF + 4 * NBLK + 1
    wait_write(0); build_fire(i_a, 0); wait_gather(3); fire_write(i_a - 1, 3)
    wait_write(1); build_fire(i_b, 1); wait_gather(0); fire_write(i_a, 0)

    @pl.when(nsteps == MAX_STEPS_W)
    def _():
        wait_write(2)
        build_fire(i_b + 1, 2)
        wait_gather(1); fire_write(i_b, 1)
        wait_gather(2); fire_write(i_b + 1, 2)

    @pl.when(nsteps == STEPS_LO)
    def _():
        wait_gather(1); fire_write(i_b, 1)

    for b in range(NBUF):
        wait_write(b)


def kernel(inp, inds):
    inp_flat = inp.reshape(NODES, F)
    inds_flat = inds.reshape(NODES * K)
    out = _gather_kernel(inp_flat, inds_flat)
    return out.reshape(B, N, KP1, F)


# speedup vs baseline: 1.5259x; 1.5259x over previous
import functools

import jax
import jax.numpy as jnp
from jax import lax
from jax.experimental import pallas as pl
from jax.experimental.pallas import tpu as pltpu
from jax.experimental.pallas import tpu_sc as plsc

B, N, F, K = 2, 10000, 128, 16
KP1 = K + 1
NODES = B * N

_info = plsc.get_sparse_core_info()
NC, NS = _info.num_cores, _info.num_subcores
NW = NC * NS

NPS = 8
STEPS = NODES // NPS
STEPS_LO = STEPS // NW
EXTRA = STEPS - STEPS_LO * NW
MAX_STEPS_W = STEPS_LO + 1
ISTRIDE = 24
NBUF = 4
IBUF = NPS * ISTRIDE

NBLK = (STEPS_LO - 2 - NBUF) // NBUF

_mesh = plsc.VectorSubcoreMesh(core_axis_name="c", subcore_axis_name="s")


@functools.partial(
    pl.kernel,
    mesh=_mesh,
    out_type=jax.ShapeDtypeStruct((NODES, KP1, F), jnp.float32),
    scratch_types=[
        pltpu.VMEM((MAX_STEPS_W * NPS * K,), jnp.int32),
        pltpu.VMEM((NBUF * IBUF + 16,), jnp.int32),
        pltpu.VMEM((NBUF, NPS, KP1, F), jnp.float32),
        pltpu.SemaphoreType.DMA,
        pltpu.SemaphoreType.DMA,
        pltpu.SemaphoreType.DMA,
        pltpu.SemaphoreType.DMA,
        pltpu.SemaphoreType.DMA,
        pltpu.SemaphoreType.DMA,
        pltpu.SemaphoreType.DMA,
        pltpu.SemaphoreType.DMA,
    ],
)
def _gather_kernel(inp_hbm, inds_hbm, out_hbm, inds_v, idx_v, rows_v,
                   sg0, sg1, sg2, sg3, sw0, sw1, sw2, sw3):
    sem_g = (sg0, sg1, sg2, sg3)
    sem_w = (sw0, sw1, sw2, sw3)

    wid = lax.axis_index("s") * NC + lax.axis_index("c")
    nsteps = STEPS_LO + jnp.where(wid < EXTRA, 1, 0)
    s0 = STEPS_LO * wid + jnp.minimum(wid, EXTRA)

    stage_s0 = jnp.minimum(s0, STEPS - MAX_STEPS_W)
    pltpu.sync_copy(
        inds_hbm.at[pl.ds(stage_s0 * (NPS * K), MAX_STEPS_W * NPS * K)],
        inds_v)
    ioff0 = (s0 - stage_s0) * (NPS * K)

    zeros = jnp.zeros((16,), jnp.int32)

    def build_fire(i, b):
        s = s0 + i
        node0 = s * NPS
        base = jnp.where(node0 >= N, N, 0).astype(jnp.int32)
        ioff = ioff0 + i * (NPS * K)
        boff = b * IBUF
        for g in range(NPS):
            idx_v[pl.ds(boff + g * ISTRIDE, K)] = (
                inds_v[pl.ds(ioff + g * K, K)] + base)
            idx_v[pl.ds(boff + g * ISTRIDE + K, 16)] = (
                zeros + (node0 + base + g))
        for g in range(NPS):
            pltpu.async_copy(
                inp_hbm.at[idx_v.at[pl.ds(boff + g * ISTRIDE, KP1)]],
                rows_v.at[b].at[g], sem_g[b])

    def wait_gather(b):
        boff = b * IBUF
        for g in range(NPS):
            pltpu.make_async_copy(
                inp_hbm.at[idx_v.at[pl.ds(boff + g * ISTRIDE, KP1)]],
                rows_v.at[b].at[g], sem_g[b]).wait()

    def fire_write(i, b):
        node0 = (s0 + i) * NPS
        pltpu.async_copy(rows_v.at[b], out_hbm.at[pl.ds(node0, NPS)],
                         sem_w[b])

    def wait_write(b):
        pltpu.make_async_copy(rows_v.at[b], out_hbm.at[pl.ds(0, NPS)],
                              sem_w[b]).wait()

    build_fire(0, 0)
    for i in range(1, NBUF):
        build_fire(i, i)
        wait_gather(i - 1)
        fire_write(i - 1, i - 1)

    def block(p, carry):
        i0 = NBUF + p * NBUF
        for j in range(NBUF):
            i = i0 + j
            bp = (j + NBUF - 1) % NBUF
            wait_write(j)
            build_fire(i, j)
            wait_gather(bp)
            fire_write(i - 1, bp)
        return carry

    lax.fori_loop(0, NBLK, block, 0)

    i_a, i_b = NBUF + 4 * NBLK, NBUF + 4 * NBLK + 1
    wait_write(0); build_fire(i_a, 0); wait_gather(3); fire_write(i_a - 1, 3)
    wait_write(1); build_fire(i_b, 1); wait_gather(0); fire_write(i_a, 0)

    @pl.when(nsteps == MAX_STEPS_W)
    def _():
        wait_write(2)
        build_fire(i_b + 1, 2)
        wait_gather(1); fire_write(i_b, 1)
        wait_gather(2); fire_write(i_b + 1, 2)

    @pl.when(nsteps == STEPS_LO)
    def _():
        wait_gather(1); fire_write(i_b, 1)

    for b in range(NBUF):
        wait_write(b)


def kernel(inp, inds):
    inp_flat = inp.reshape(NODES, F)
    inds_flat = inds.reshape(NODES * K)
    out = _gather_kernel(inp_flat, inds_flat)
    return out.reshape(B, N, KP1, F)


# speedup vs baseline: 1.5275x; 1.0010x over previous
import functools

import jax
import jax.numpy as jnp
from jax import lax
from jax.experimental import pallas as pl
from jax.experimental.pallas import tpu as pltpu
from jax.experimental.pallas import tpu_sc as plsc

B, N, F, K = 2, 10000, 128, 16
KP1 = K + 1
NODES = B * N

_info = plsc.get_sparse_core_info()
NC, NS = _info.num_cores, _info.num_subcores
NW = NC * NS

NPS = 16
STEPS = NODES // NPS
STEPS_LO = STEPS // NW
EXTRA = STEPS - STEPS_LO * NW
MAX_STEPS_W = STEPS_LO + 1
ISTRIDE = 24
NBUF = 2
IBUF = NPS * ISTRIDE

NBLK = (STEPS_LO - NBUF - 1) // NBUF
PEEL = STEPS_LO - NBUF - NBUF * NBLK

_mesh = plsc.VectorSubcoreMesh(core_axis_name="c", subcore_axis_name="s")


@functools.partial(
    pl.kernel,
    mesh=_mesh,
    out_type=jax.ShapeDtypeStruct((NODES, KP1, F), jnp.float32),
    scratch_types=[
        pltpu.VMEM((MAX_STEPS_W * NPS * K,), jnp.int32),
        pltpu.VMEM((NBUF * IBUF + 16,), jnp.int32),
        pltpu.VMEM((NBUF, NPS, KP1, F), jnp.float32),
        pltpu.SemaphoreType.DMA,
        pltpu.SemaphoreType.DMA,
        pltpu.SemaphoreType.DMA,
        pltpu.SemaphoreType.DMA,
        pltpu.SemaphoreType.DMA,
        pltpu.SemaphoreType.DMA,
        pltpu.SemaphoreType.DMA,
        pltpu.SemaphoreType.DMA,
    ],
)
def _gather_kernel(inp_hbm, inds_hbm, out_hbm, inds_v, idx_v, rows_v,
                   sg0, sg1, sg2, sg3, sw0, sw1, sw2, sw3):
    sem_g = (sg0, sg1, sg2, sg3)
    sem_w = (sw0, sw1, sw2, sw3)

    wid = lax.axis_index("s") * NC + lax.axis_index("c")
    nsteps = STEPS_LO + jnp.where(wid < EXTRA, 1, 0)
    s0 = STEPS_LO * wid + jnp.minimum(wid, EXTRA)

    stage_s0 = jnp.minimum(s0, STEPS - MAX_STEPS_W)
    pltpu.sync_copy(
        inds_hbm.at[pl.ds(stage_s0 * (NPS * K), MAX_STEPS_W * NPS * K)],
        inds_v)
    ioff0 = (s0 - stage_s0) * (NPS * K)

    zeros = jnp.zeros((16,), jnp.int32)

    def build_fire(i, b):
        s = s0 + i
        node0 = s * NPS
        base = jnp.where(node0 >= N, N, 0).astype(jnp.int32)
        ioff = ioff0 + i * (NPS * K)
        boff = b * IBUF
        for g in range(NPS):
            idx_v[pl.ds(boff + g * ISTRIDE, K)] = (
                inds_v[pl.ds(ioff + g * K, K)] + base)
            idx_v[pl.ds(boff + g * ISTRIDE + K, 16)] = (
                zeros + (node0 + base + g))
        for g in range(NPS):
            pltpu.async_copy(
                inp_hbm.at[idx_v.at[pl.ds(boff + g * ISTRIDE, KP1)]],
                rows_v.at[b].at[g], sem_g[b])

    def wait_gather(b):
        boff = b * IBUF
        for g in range(NPS):
            pltpu.make_async_copy(
                inp_hbm.at[idx_v.at[pl.ds(boff + g * ISTRIDE, KP1)]],
                rows_v.at[b].at[g], sem_g[b]).wait()

    def fire_write(i, b):
        node0 = (s0 + i) * NPS
        pltpu.async_copy(rows_v.at[b], out_hbm.at[pl.ds(node0, NPS)],
                         sem_w[b])

    def wait_write(b):
        pltpu.make_async_copy(rows_v.at[b], out_hbm.at[pl.ds(0, NPS)],
                              sem_w[b]).wait()

    build_fire(0, 0)
    for i in range(1, NBUF):
        build_fire(i, i)
        wait_gather(i - 1)
        fire_write(i - 1, i - 1)

    def block(p, carry):
        i0 = NBUF + p * NBUF
        for j in range(NBUF):
            i = i0 + j
            bp = (j + NBUF - 1) % NBUF
            wait_write(j)
            build_fire(i, j)
            wait_gather(bp)
            fire_write(i - 1, bp)
        return carry

    lax.fori_loop(0, NBLK, block, 0)

    for t in range(PEEL):
        i = NBUF + NBUF * NBLK + t
        bb = i % NBUF
        bp = (bb + NBUF - 1) % NBUF
        wait_write(bb); build_fire(i, bb)
        wait_gather(bp); fire_write(i - 1, bp)
    i_last = STEPS_LO - 1
    b_last = i_last % NBUF
    b_xtra = (i_last + 1) % NBUF

    @pl.when(nsteps == MAX_STEPS_W)
    def _():
        wait_write(b_xtra)
        build_fire(i_last + 1, b_xtra)
        wait_gather(b_last); fire_write(i_last, b_last)
        wait_gather(b_xtra); fire_write(i_last + 1, b_xtra)

    @pl.when(nsteps == STEPS_LO)
    def _():
        wait_gather(b_last); fire_write(i_last, b_last)

    for b in range(NBUF):
        wait_write(b)


def kernel(inp, inds):
    inp_flat = inp.reshape(NODES, F)
    inds_flat = inds.reshape(NODES * K)
    out = _gather_kernel(inp_flat, inds_flat)
    return out.reshape(B, N, KP1, F)
